# Initial kernel scaffold; baseline (speedup 1.0000x reference)
import jax
import jax.numpy as jnp
from jax.experimental import pallas as pl


def kernel(x, edge_index, W_edge, b_edge, W_proj, b_proj, W_ih, W_hh, b_ih, b_hh):
    raise NotImplementedError("write your pallas kernel here")


# speedup vs baseline: 15.1643x; 15.1643x over previous
import functools

import jax
import jax.numpy as jnp
from jax import lax
from jax.experimental import pallas as pl
from jax.experimental.pallas import tpu as pltpu
from jax.experimental.pallas import tpu_sc as plsc

N = 10000
NPAD = 10240
E = 320000
D = 128
G = 128

NC, NS, L = 2, 16, 16
NW = NC * NS
EPW = E // NW
C = 80
NCHUNK = EPW // C
RPT = NPAD // NS


def _pre_body(x_ref, wproj_ref, bproj_ref, wds_ref, hv_ref, pdps_ref):
    x = x_ref[...]
    hv_ref[...] = jnp.dot(x, wproj_ref[...],
                          preferred_element_type=jnp.float32) + bproj_ref[...]
    pdps_ref[...] = jnp.dot(x, wds_ref[...],
                            preferred_element_type=jnp.float32)


def _pre_call(x_pad, W_proj, b_proj, wds):
    blk = 1024
    grid = NPAD // blk
    return pl.pallas_call(
        _pre_body,
        grid=(grid,),
        in_specs=[
            pl.BlockSpec((blk, D), lambda i: (i, 0)),
            pl.BlockSpec((D, G), lambda i: (0, 0)),
            pl.BlockSpec((1, G), lambda i: (0, 0)),
            pl.BlockSpec((D, 8), lambda i: (0, 0)),
        ],
        out_specs=[
            pl.BlockSpec((blk, G), lambda i: (i, 0)),
            pl.BlockSpec((blk, 8), lambda i: (i, 0)),
        ],
        out_shape=[
            jax.ShapeDtypeStruct((NPAD, G), jnp.float32),
            jax.ShapeDtypeStruct((NPAD, 8), jnp.float32),
        ],
    )(x_pad, W_proj, b_proj, wds)


def _sc_body(src_hbm, dst_hbm, pd_hbm, ps_hbm, hv_hbm, c_out, s_out,
             pd_v, ps_v, src_v, dst_v, s_loc, rows_v, sem,
             c_sh):
    core = lax.axis_index("c")
    sub = lax.axis_index("s")
    wid = sub * NC + core
    tid = sub

    pltpu.sync_copy(pd_hbm, pd_v)
    pltpu.sync_copy(ps_hbm, ps_v)

    zvec = jnp.zeros((L,), jnp.float32)

    def _zero_row(r, _):
        for k in range(8):
            rows_v[r, k * L:(k + 1) * L] = zvec
        return ()

    lax.fori_loop(0, C, _zero_row, ())

    def _zero_s(j, _):
        s_loc[pl.ds(j * L, L)] = zvec
        return ()

    lax.fori_loop(0, NPAD // L, _zero_s, ())

    for j in range(RPT // C):
        pltpu.sync_copy(rows_v, c_sh.at[pl.ds(tid * RPT + j * C, C)])
    plsc.subcore_barrier()

    def _chunk(i, _):
        base = wid * EPW + i * C
        pltpu.sync_copy(src_hbm.at[pl.ds(base, C)], src_v)
        pltpu.sync_copy(dst_hbm.at[pl.ds(base, C)], dst_v)
        pltpu.async_copy(hv_hbm.at[src_v], rows_v, sem).wait()

        def _group(g, _):
            idx_d = dst_v[pl.ds(g * L, L)]
            idx_s = src_v[pl.ds(g * L, L)]
            t = plsc.load_gather(pd_v, [idx_d]) + plsc.load_gather(ps_v, [idx_s])
            w = jnp.exp(jnp.maximum(t, 0.01 * t))
            plsc.addupdate_scatter(s_loc, [idx_d], w)
            for e in range(L):
                we = lax.broadcast_in_dim(w[e], (L,), ())
                row = g * L + e
                for k in range(8):
                    rows_v[row, k * L:(k + 1) * L] = (
                        rows_v[row, k * L:(k + 1) * L] * we)
            return ()

        lax.fori_loop(0, C // L, _group, ())

        pltpu.sync_copy(rows_v, c_sh.at[dst_v], add=True)
        return ()

    lax.fori_loop(0, NCHUNK, _chunk, ())
    plsc.subcore_barrier()

    r0 = tid * RPT
    pltpu.sync_copy(c_sh.at[pl.ds(r0, RPT)], c_out.at[core, pl.ds(r0, RPT)])
    pltpu.sync_copy(s_loc, s_out.at[core, tid])


def _sc_call(src, dst, pd, ps, hv):
    mesh = plsc.VectorSubcoreMesh(core_axis_name="c", subcore_axis_name="s")
    f = pl.kernel(
        _sc_body,
        compiler_params=pltpu.CompilerParams(needs_layout_passes=False),
        out_type=[
            jax.ShapeDtypeStruct((NC, NPAD, G), jnp.float32),
            jax.ShapeDtypeStruct((NC, NS, NPAD), jnp.float32),
        ],
        mesh=mesh,
        scratch_types=[
            pltpu.VMEM((NPAD,), jnp.float32),
            pltpu.VMEM((NPAD,), jnp.float32),
            pltpu.VMEM((C,), jnp.int32),
            pltpu.VMEM((C,), jnp.int32),
            pltpu.VMEM((NPAD,), jnp.float32),
            pltpu.VMEM((C, G), jnp.float32),
            pltpu.SemaphoreType.DMA,
            pltpu.VMEM_SHARED((NPAD, G), jnp.float32),
        ],
    )
    return f(src, dst, pd, ps, hv)


def _post_body(x_ref, c0_ref, c1_ref, s_ref, wih_ref, bih_ref,
               whh_ref, bhh_ref, out_ref):
    x = x_ref[...]
    s = jnp.sum(s_ref[...], axis=0)[:, None]
    rs = 1.0 / jnp.maximum(s, 1e-30)
    cc = (c0_ref[...] + c1_ref[...]) * rs
    ctx = jnp.where(cc > 0, cc, jnp.exp(cc) - 1.0)
    gi = jnp.dot(ctx, wih_ref[...], preferred_element_type=jnp.float32) + bih_ref[...]
    gh = jnp.dot(x, whh_ref[...], preferred_element_type=jnp.float32) + bhh_ref[...]
    r = jax.nn.sigmoid(gi[:, 0:D] + gh[:, 0:D])
    z = jax.nn.sigmoid(gi[:, D:2 * D] + gh[:, D:2 * D])
    n = jnp.tanh(gi[:, 2 * D:3 * D] + r * gh[:, 2 * D:3 * D])
    h = (1.0 - z) * n + z * x
    out_ref[...] = jnp.maximum(h, 0.0)


def _post_call(x_pad, c0, c1, s32, wihT, b_ih, whhT, b_hh):
    blk = 1024
    grid = NPAD // blk
    return pl.pallas_call(
        _post_body,
        grid=(grid,),
        in_specs=[
            pl.BlockSpec((blk, D), lambda i: (i, 0)),
            pl.BlockSpec((blk, G), lambda i: (i, 0)),
            pl.BlockSpec((blk, G), lambda i: (i, 0)),
            pl.BlockSpec((NW, blk), lambda i: (0, i)),
            pl.BlockSpec((G, 3 * D), lambda i: (0, 0)),
            pl.BlockSpec((1, 3 * D), lambda i: (0, 0)),
            pl.BlockSpec((D, 3 * D), lambda i: (0, 0)),
            pl.BlockSpec((1, 3 * D), lambda i: (0, 0)),
        ],
        out_specs=pl.BlockSpec((blk, D), lambda i: (i, 0)),
        out_shape=jax.ShapeDtypeStruct((NPAD, D), jnp.float32),
    )(x_pad, c0, c1, s32, wihT, b_ih, whhT, b_hh)


def kernel(x, edge_index, W_edge, b_edge, W_proj, b_proj, W_ih, W_hh, b_ih, b_hh):
    x_pad = jnp.pad(x, ((0, NPAD - N), (0, 0)))
    wds = jnp.concatenate(
        [W_edge[:D], W_edge[D:], jnp.zeros((D, 6), jnp.float32)], axis=1)
    hv, pdps = _pre_call(x_pad, W_proj, b_proj.reshape(1, G), wds)
    pd = pdps[:, 0] + b_edge[0]
    ps = pdps[:, 1]

    c_out, s_out = _sc_call(edge_index[0], edge_index[1], pd, ps, hv)

    out = _post_call(
        x_pad, c_out[0], c_out[1], s_out.reshape(NW, NPAD),
        W_ih.T, b_ih.reshape(1, 3 * D), W_hh.T, b_hh.reshape(1, 3 * D))
    return out[:N]
